# baseline (speedup 1.0000x reference)
import functools

import jax
import jax.numpy as jnp
from jax import lax
from jax.experimental import pallas as pl
from jax.experimental.pallas import tpu as pltpu
from jax.experimental.pallas import tpu_sc as plsc

_NUM_CORES = 2
_NUM_SUBCORES = 16
_NUM_WORKERS = _NUM_CORES * _NUM_SUBCORES

_CHUNK = 32
_LANES = 16


def _emb_kernel(S, D, rows_per_worker,
                ids_hbm, tids_hbm, tok_hbm, pos_hbm, task_hbm, out_hbm,
                idx_v, tidx_v, buf, tokbuf, taskbuf, sem):
  wid = lax.axis_index("s") * _NUM_CORES + lax.axis_index("c")
  base = wid * rows_per_worker
  workers_per_batch = max(S // rows_per_worker, 1)
  ndg = D // _LANES

  pltpu.sync_copy(tids_hbm.at[pl.ds(base, 8)], tidx_v)
  pltpu.async_copy(task_hbm.at[tidx_v], taskbuf, sem).wait()

  for c in range(rows_per_worker // _CHUNK):
    off = base + c * _CHUNK
    s0 = (wid % workers_per_batch) * rows_per_worker + c * _CHUNK
    pltpu.sync_copy(ids_hbm.at[pl.ds(off, _CHUNK)], idx_v)
    cp = pltpu.async_copy(tok_hbm.at[idx_v], tokbuf, sem)
    pltpu.sync_copy(pos_hbm.at[pl.ds(s0, _CHUNK)], buf)
    cp.wait()
    for half in range(2):
      dg0 = half * (ndg // 2)
      tvs = [taskbuf[0, pl.ds((dg0 + dg) * _LANES, _LANES)]
             for dg in range(ndg // 2)]

      def r_body(r, carry, dg0=dg0, tvs=tvs):
        for dg in range(ndg // 2):
          dgs = pl.ds((dg0 + dg) * _LANES, _LANES)
          plsc.addupdate(buf.at[r, dgs], tokbuf[r, dgs] + tvs[dg])
        return carry

      lax.fori_loop(0, _CHUNK, r_body, 0)
    pltpu.sync_copy(buf, out_hbm.at[pl.ds(off, _CHUNK)])


@jax.jit
def kernel(input_ids, task_ids, token_table, pos_table, task_table):
  B, S = input_ids.shape
  V, D = token_table.shape
  N = B * S
  rows_per_worker = N // _NUM_WORKERS

  flat_ids = jnp.asarray(input_ids, jnp.int32).reshape(N)
  flat_tids = jnp.repeat(jnp.asarray(task_ids, jnp.int32), S)

  mesh = plsc.VectorSubcoreMesh(core_axis_name="c", subcore_axis_name="s")
  body = functools.partial(_emb_kernel, S, D, rows_per_worker)
  out = pl.kernel(
      body,
      out_type=jax.ShapeDtypeStruct((N, D), jnp.float32),
      mesh=mesh,
      scratch_types=[
          pltpu.VMEM((_CHUNK,), jnp.int32),
          pltpu.VMEM((8,), jnp.int32),
          pltpu.VMEM((_CHUNK, D), jnp.float32),
          pltpu.VMEM((_CHUNK, D), jnp.float32),
          pltpu.VMEM((8, D), jnp.float32),
          pltpu.SemaphoreType.DMA,
      ],
  )(flat_ids, flat_tids, token_table, pos_table, task_table)
  return out.reshape(B, S, D)


# speedup vs baseline: 1.1173x; 1.1173x over previous
import functools

import jax
import jax.numpy as jnp
from jax import lax
from jax.experimental import pallas as pl
from jax.experimental.pallas import tpu as pltpu
from jax.experimental.pallas import tpu_sc as plsc

_NUM_CORES = 2
_NUM_SUBCORES = 16
_NUM_WORKERS = _NUM_CORES * _NUM_SUBCORES

_C = 16
_LANES = 16


def _emb_kernel(B, S, D,
                ids_hbm, tids_hbm, tok_hbm, pos_hbm, task_hbm, out_hbm,
                idx2d, tidx_v, posbuf, tok_a, tok_b, taskbuf,
                sem_tok_a, sem_tok_b, sem_pos, sem_out_a, sem_out_b,
                sem_task):
  srange = S // _NUM_WORKERS
  wid = lax.axis_index("s") * _NUM_CORES + lax.axis_index("c")
  sbase = wid * srange
  chunks_per_b = srange // _C
  nchunks = B * chunks_per_b
  ndg = D // _LANES

  for b in range(B):
    pltpu.sync_copy(ids_hbm.at[pl.ds(b * S + sbase, srange)], idx2d.at[b])
  pltpu.sync_copy(tids_hbm, tidx_v)
  taskcp = pltpu.async_copy(task_hbm.at[tidx_v], taskbuf, sem_task)
  poscp = pltpu.async_copy(pos_hbm.at[pl.ds(sbase, srange)], posbuf, sem_pos)

  toks = [tok_a, tok_b]
  sem_toks = [sem_tok_a, sem_tok_b]
  sem_outs = [sem_out_a, sem_out_b]

  def start_gather(k):
    b, h = k // chunks_per_b, k % chunks_per_b
    return pltpu.async_copy(
        tok_hbm.at[idx2d.at[b, pl.ds(h * _C, _C)]], toks[k % 2],
        sem_toks[k % 2])

  gcp = [None] * nchunks
  ocp = [None] * nchunks
  gcp[0] = start_gather(0)
  taskcp.wait()
  poscp.wait()

  for k in range(nchunks):
    b, h = k // chunks_per_b, k % chunks_per_b
    cur = k % 2
    if k + 1 < nchunks:
      if k >= 1:
        ocp[k - 1].wait()
      gcp[k + 1] = start_gather(k + 1)
    gcp[k].wait()
    tok = toks[cur]
    for half in range(2):
      dg0 = half * (ndg // 2)
      tvs = [taskbuf[b, pl.ds((dg0 + dg) * _LANES, _LANES)]
             for dg in range(ndg // 2)]

      def r_body(r, carry, h=h, tok=tok, dg0=dg0, tvs=tvs):
        pr = h * _C + r
        for dg in range(len(tvs)):
          dgs = pl.ds((dg0 + dg) * _LANES, _LANES)
          plsc.addupdate(tok.at[r, dgs], posbuf[pr, dgs] + tvs[dg])
        return carry

      lax.fori_loop(0, _C, r_body, 0)
    ocp[k] = pltpu.async_copy(
        tok, out_hbm.at[pl.ds(b * S + sbase + h * _C, _C)], sem_outs[cur])

  ocp[nchunks - 2].wait()
  ocp[nchunks - 1].wait()


@jax.jit
def kernel(input_ids, task_ids, token_table, pos_table, task_table):
  B, S = input_ids.shape
  V, D = token_table.shape
  N = B * S
  srange = S // _NUM_WORKERS

  flat_ids = jnp.asarray(input_ids, jnp.int32).reshape(N)
  tids8 = jnp.concatenate([jnp.asarray(task_ids, jnp.int32)] * (8 // B))

  mesh = plsc.VectorSubcoreMesh(core_axis_name="c", subcore_axis_name="s")
  body = functools.partial(_emb_kernel, B, S, D)
  out = pl.kernel(
      body,
      out_type=jax.ShapeDtypeStruct((N, D), jnp.float32),
      mesh=mesh,
      scratch_types=[
          pltpu.VMEM((B, srange), jnp.int32),
          pltpu.VMEM((8,), jnp.int32),
          pltpu.VMEM((srange, D), jnp.float32),
          pltpu.VMEM((_C, D), jnp.float32),
          pltpu.VMEM((_C, D), jnp.float32),
          pltpu.VMEM((8, D), jnp.float32),
          pltpu.SemaphoreType.DMA,
          pltpu.SemaphoreType.DMA,
          pltpu.SemaphoreType.DMA,
          pltpu.SemaphoreType.DMA,
          pltpu.SemaphoreType.DMA,
          pltpu.SemaphoreType.DMA,
      ],
  )(flat_ids, tids8, token_table, pos_table, task_table)
  return out.reshape(B, S, D)


# speedup vs baseline: 1.9145x; 1.7135x over previous
import functools

import jax
import jax.numpy as jnp
from jax import lax
from jax.experimental import pallas as pl
from jax.experimental.pallas import tpu as pltpu
from jax.experimental.pallas import tpu_sc as plsc

_NUM_CORES = 2
_NUM_SUBCORES = 16
_NUM_WORKERS = _NUM_CORES * _NUM_SUBCORES

_C = 16
_LANES = 16


def _emb_kernel(B, S, D,
                ids_hbm, tids_hbm, tok_hbm, pos_hbm, task_hbm, out_hbm,
                idx2d, tidx_v, posbuf, tok_a, tok_b, taskbuf,
                sem_tok_a, sem_tok_b, sem_pos, sem_out_a, sem_out_b,
                sem_task):
  srange = S // _NUM_WORKERS
  wid = lax.axis_index("s") * _NUM_CORES + lax.axis_index("c")
  sbase = wid * srange
  chunks_per_b = srange // _C
  nchunks = B * chunks_per_b
  ndg = D // _LANES

  for b in range(B):
    pltpu.sync_copy(ids_hbm.at[pl.ds(b * S + sbase, srange)], idx2d.at[b])
  pltpu.sync_copy(tids_hbm, tidx_v)
  taskcp = pltpu.async_copy(task_hbm.at[tidx_v], taskbuf, sem_task)
  poscp = pltpu.async_copy(pos_hbm.at[pl.ds(sbase, srange)], posbuf, sem_pos)

  toks = [tok_a, tok_b]
  sem_toks = [sem_tok_a, sem_tok_b]
  sem_outs = [sem_out_a, sem_out_b]

  def start_gather(k):
    b, h = k // chunks_per_b, k % chunks_per_b
    return pltpu.async_copy(
        tok_hbm.at[idx2d.at[b, pl.ds(h * _C, _C)]], toks[k % 2],
        sem_toks[k % 2])

  gcp = [None] * nchunks
  ocp = [None] * nchunks
  gcp[0] = start_gather(0)
  taskcp.wait()
  poscp.wait()

  for k in range(nchunks):
    b, h = k // chunks_per_b, k % chunks_per_b
    cur = k % 2
    if k + 1 < nchunks:
      if k >= 1:
        ocp[k - 1].wait()
      gcp[k + 1] = start_gather(k + 1)
    gcp[k].wait()
    tok = toks[cur]
    for half in range(0):
      dg0 = half * (ndg // 2)
      tvs = [taskbuf[b, pl.ds((dg0 + dg) * _LANES, _LANES)]
             for dg in range(ndg // 2)]

      def r_body(r, carry, h=h, tok=tok, dg0=dg0, tvs=tvs):
        pr = h * _C + r
        for dg in range(len(tvs)):
          dgs = pl.ds((dg0 + dg) * _LANES, _LANES)
          plsc.addupdate(tok.at[r, dgs], posbuf[pr, dgs] + tvs[dg])
        return carry

      lax.fori_loop(0, _C, r_body, 0)
    ocp[k] = pltpu.async_copy(
        tok, out_hbm.at[pl.ds(b * S + sbase + h * _C, _C)], sem_outs[cur])

  ocp[nchunks - 2].wait()
  ocp[nchunks - 1].wait()


@jax.jit
def kernel(input_ids, task_ids, token_table, pos_table, task_table):
  B, S = input_ids.shape
  V, D = token_table.shape
  N = B * S
  srange = S // _NUM_WORKERS

  flat_ids = jnp.asarray(input_ids, jnp.int32).reshape(N)
  tids8 = jnp.concatenate([jnp.asarray(task_ids, jnp.int32)] * (8 // B))

  mesh = plsc.VectorSubcoreMesh(core_axis_name="c", subcore_axis_name="s")
  body = functools.partial(_emb_kernel, B, S, D)
  out = pl.kernel(
      body,
      out_type=jax.ShapeDtypeStruct((N, D), jnp.float32),
      mesh=mesh,
      scratch_types=[
          pltpu.VMEM((B, srange), jnp.int32),
          pltpu.VMEM((8,), jnp.int32),
          pltpu.VMEM((srange, D), jnp.float32),
          pltpu.VMEM((_C, D), jnp.float32),
          pltpu.VMEM((_C, D), jnp.float32),
          pltpu.VMEM((8, D), jnp.float32),
          pltpu.SemaphoreType.DMA,
          pltpu.SemaphoreType.DMA,
          pltpu.SemaphoreType.DMA,
          pltpu.SemaphoreType.DMA,
          pltpu.SemaphoreType.DMA,
          pltpu.SemaphoreType.DMA,
      ],
  )(flat_ids, tids8, token_table, pos_table, task_table)
  return out.reshape(B, S, D)


# speedup vs baseline: 1.9731x; 1.0307x over previous
import functools

import jax
import jax.numpy as jnp
from jax import lax
from jax.experimental import pallas as pl
from jax.experimental.pallas import tpu as pltpu
from jax.experimental.pallas import tpu_sc as plsc

_NUM_CORES = 2
_NUM_SUBCORES = 16
_NUM_WORKERS = _NUM_CORES * _NUM_SUBCORES

_C = 16
_LANES = 16


def _emb_kernel(B, S, D,
                ids_hbm, tids_hbm, tok_hbm, pos_hbm, task_hbm, out_hbm,
                idx2d, tidx_v, posbuf, tok_a, tok_b, tok_c, taskbuf,
                sem_tok_a, sem_tok_b, sem_tok_c, sem_pos,
                sem_out_a, sem_out_b, sem_out_c, sem_task):
  srange = S // _NUM_WORKERS
  wid = lax.axis_index("s") * _NUM_CORES + lax.axis_index("c")
  sbase = wid * srange
  chunks_per_b = srange // _C
  nchunks = B * chunks_per_b
  ndg = D // _LANES

  pltpu.sync_copy(ids_hbm.at[pl.ds(sbase, srange)], idx2d.at[0])

  toks = [tok_a, tok_b, tok_c]
  sem_toks = [sem_tok_a, sem_tok_b, sem_tok_c]
  sem_outs = [sem_out_a, sem_out_b, sem_out_c]
  nbuf = len(toks)

  def start_gather(k):
    b, h = k // chunks_per_b, k % chunks_per_b
    return pltpu.async_copy(
        tok_hbm.at[idx2d.at[b, pl.ds(h * _C, _C)]], toks[k % nbuf],
        sem_toks[k % nbuf])

  gcp = [None] * nchunks
  ocp = [None] * nchunks
  gcp[0] = start_gather(0)

  for b in range(1, B):
    pltpu.sync_copy(ids_hbm.at[pl.ds(b * S + sbase, srange)], idx2d.at[b])
  pltpu.sync_copy(tids_hbm, tidx_v)
  taskcp = pltpu.async_copy(task_hbm.at[tidx_v], taskbuf, sem_task)
  poscp = pltpu.async_copy(pos_hbm.at[pl.ds(sbase, srange)], posbuf, sem_pos)

  gcp[1] = start_gather(1)
  taskcp.wait()
  poscp.wait()

  for k in range(nchunks):
    b, h = k // chunks_per_b, k % chunks_per_b
    cur = k % nbuf
    if k + 2 < nchunks:
      if k >= 1:
        ocp[k - 1].wait()
      gcp[k + 2] = start_gather(k + 2)
    gcp[k].wait()
    tok = toks[cur]
    for half in range(2):
      dg0 = half * (ndg // 2)
      tvs = [taskbuf[b, pl.ds((dg0 + dg) * _LANES, _LANES)]
             for dg in range(ndg // 2)]

      @functools.partial(plsc.parallel_loop, 0, _C, unroll=2)
      def r_body(r, h=h, tok=tok, dg0=dg0, tvs=tvs):
        pr = h * _C + r
        for dg in range(len(tvs)):
          dgs = pl.ds((dg0 + dg) * _LANES, _LANES)
          plsc.addupdate(tok.at[r, dgs], posbuf[pr, dgs] + tvs[dg])

    ocp[k] = pltpu.async_copy(
        tok, out_hbm.at[pl.ds(b * S + sbase + h * _C, _C)], sem_outs[cur])

  for k in range(max(nchunks - nbuf, 0), nchunks):
    ocp[k].wait()


@jax.jit
def kernel(input_ids, task_ids, token_table, pos_table, task_table):
  B, S = input_ids.shape
  V, D = token_table.shape
  N = B * S
  srange = S // _NUM_WORKERS

  flat_ids = jnp.asarray(input_ids, jnp.int32).reshape(N)
  tids8 = jnp.concatenate([jnp.asarray(task_ids, jnp.int32)] * (8 // B))

  mesh = plsc.VectorSubcoreMesh(core_axis_name="c", subcore_axis_name="s")
  body = functools.partial(_emb_kernel, B, S, D)
  out = pl.kernel(
      body,
      out_type=jax.ShapeDtypeStruct((N, D), jnp.float32),
      mesh=mesh,
      scratch_types=[
          pltpu.VMEM((B, srange), jnp.int32),
          pltpu.VMEM((8,), jnp.int32),
          pltpu.VMEM((srange, D), jnp.float32),
          pltpu.VMEM((_C, D), jnp.float32),
          pltpu.VMEM((_C, D), jnp.float32),
          pltpu.VMEM((_C, D), jnp.float32),
          pltpu.VMEM((8, D), jnp.float32),
          pltpu.SemaphoreType.DMA,
          pltpu.SemaphoreType.DMA,
          pltpu.SemaphoreType.DMA,
          pltpu.SemaphoreType.DMA,
          pltpu.SemaphoreType.DMA,
          pltpu.SemaphoreType.DMA,
          pltpu.SemaphoreType.DMA,
          pltpu.SemaphoreType.DMA,
      ],
  )(flat_ids, tids8, token_table, pos_table, task_table)
  return out.reshape(B, S, D)
